# baseline (speedup 1.0000x reference)
import math

import jax
import jax.numpy as jnp
from jax import lax
from jax.experimental import pallas as pl
from jax.experimental.pallas import tpu as pltpu

B, S, D = 2, 2048, 1024
N, HID, HD, G = 16, 64, 64, 64
T = B * S
TM = 1024
SB = S // TM
PACK = 4
KP = PACK * HID

_TT = (((1,), (1,)), ((), ()))
_NT = (((1,), (0,)), ((), ()))
_TN = (((0,), (0,)), ((), ()))


def _gelu_tanh(h):
    u = h * (jnp.float32(0.7978845608028654)
             + jnp.float32(0.7978845608028654 * 0.044715) * (h * h))
    r = 0.5 * h
    return r + r * jnp.tanh(u)


def _body(temp_ref, x_ref, rw_ref, c_ref, win_ref, wout_ref, out_ref,
          win2_s, bd_s, emat_s):
    @pl.when(pl.program_id(0) == 0)
    def _prep():
        for n in range(N):
            win2_s[:, n * HID:(n + 1) * HID] = win_ref[n].astype(jnp.bfloat16)
        bd_s[:] = jnp.zeros((N // PACK, KP, KP), jnp.bfloat16)
        for g in range(N // PACK):
            for a in range(PACK):
                bd_s[g, a * HID:(a + 1) * HID, a * HD:(a + 1) * HD] = (
                    wout_ref[g * PACK + a].astype(jnp.bfloat16))
        lane = lax.broadcasted_iota(jnp.int32, (N, N * HD), 1) // HD
        sub = lax.broadcasted_iota(jnp.int32, (N, N * HD), 0)
        emat_s[:] = (lane == sub).astype(jnp.bfloat16)

    xb = x_ref[0].astype(jnp.bfloat16)

    zt = lax.dot_general(rw_ref[:].astype(jnp.bfloat16), xb, _TT,
                         preferred_element_type=jnp.float32)
    znt = zt / jnp.maximum(jnp.sqrt(jnp.sum(zt * zt, axis=0, keepdims=True)), 1e-12)
    c = c_ref[:]
    cn = c / jnp.maximum(jnp.sqrt(jnp.sum(c * c, axis=1, keepdims=True)), 1e-12)
    lt = lax.dot_general(cn.astype(jnp.bfloat16), znt.astype(jnp.bfloat16),
                         _NT, preferred_element_type=jnp.float32)
    lt = lt * (jnp.exp(temp_ref[0]) / math.sqrt(G))

    nidx = lax.broadcasted_iota(jnp.int32, (N, TM), 0)
    neg = jnp.float32(-1e30)
    m1 = jnp.max(lt, axis=0, keepdims=True)
    i1 = jnp.min(jnp.where(lt == m1, nidx, N), axis=0, keepdims=True)
    l2 = jnp.where(nidx == i1, neg, lt)
    m2 = jnp.max(l2, axis=0, keepdims=True)
    i2 = jnp.min(jnp.where(l2 == m2, nidx, N), axis=0, keepdims=True)

    e = jnp.exp(lt - m1)
    probs = e / jnp.sum(e, axis=0, keepdims=True)
    p1 = jnp.max(probs, axis=0, keepdims=True)
    p2 = jnp.max(jnp.where(nidx == i1, neg, probs), axis=0, keepdims=True)
    wt = jnp.where(nidx == i1, p1, 0.0) + jnp.where(nidx == i2, p2, 0.0)
    wexp = lax.dot_general(wt.astype(jnp.bfloat16), emat_s[:], _TN,
                           preferred_element_type=jnp.float32)

    for g in range(N // PACK):
        hidden = lax.dot_general(xb, win2_s[:, g * KP:(g + 1) * KP], _NT,
                                 preferred_element_type=jnp.float32)
        hb = _gelu_tanh(hidden).astype(jnp.bfloat16)
        out_ref[0, :, g * KP:(g + 1) * KP] = lax.dot_general(
            hb, bd_s[g], _NT,
            preferred_element_type=jnp.float32) * wexp[:, g * KP:(g + 1) * KP]


def kernel(x, router_w, head_centroids, temperature, input_experts, output_experts):
    return pl.pallas_call(
        _body,
        grid=(T // TM,),
        in_specs=[
            pl.BlockSpec(memory_space=pltpu.SMEM),
            pl.BlockSpec((1, TM, D), lambda i: (i // SB, i % SB, 0)),
            pl.BlockSpec((G, D), lambda i: (0, 0)),
            pl.BlockSpec((N, G), lambda i: (0, 0)),
            pl.BlockSpec((N, D, HID), lambda i: (0, 0, 0)),
            pl.BlockSpec((N, HID, HD), lambda i: (0, 0, 0)),
        ],
        out_specs=pl.BlockSpec((1, TM, N * HD), lambda i: (i // SB, i % SB, 0)),
        out_shape=jax.ShapeDtypeStruct((B, S, N * HD), jnp.float32),
        scratch_shapes=[
            pltpu.VMEM((D, N * HID), jnp.bfloat16),
            pltpu.VMEM((N // PACK, KP, KP), jnp.bfloat16),
            pltpu.VMEM((N, N * HD), jnp.bfloat16),
        ],
        compiler_params=pltpu.CompilerParams(dimension_semantics=("arbitrary",)),
    )(temperature, x, router_w, head_centroids, input_experts, output_experts)


# speedup vs baseline: 1.1395x; 1.1395x over previous
import math

import jax
import jax.numpy as jnp
from jax import lax
from jax.experimental import pallas as pl
from jax.experimental.pallas import tpu as pltpu

B, S, D = 2, 2048, 1024
N, HID, HD, G = 16, 64, 64, 64
T = B * S
TM = 2048
SB = S // TM
PACK = 4
KP = PACK * HID

_TT = (((1,), (1,)), ((), ()))
_NT = (((1,), (0,)), ((), ()))
_TN = (((0,), (0,)), ((), ()))


def _gelu_tanh(h):
    u = h * (jnp.float32(0.7978845608028654)
             + jnp.float32(0.7978845608028654 * 0.044715) * (h * h))
    r = 0.5 * h
    return r + r * jnp.tanh(u)


def _body(temp_ref, x_ref, rw_ref, c_ref, win_ref, wout_ref, out_ref,
          win2_s, bd_s, emat_s):
    @pl.when(pl.program_id(0) == 0)
    def _prep():
        for n in range(N):
            win2_s[:, n * HID:(n + 1) * HID] = win_ref[n].astype(jnp.bfloat16)
        bd_s[:] = jnp.zeros((N // PACK, KP, KP), jnp.bfloat16)
        for g in range(N // PACK):
            for a in range(PACK):
                bd_s[g, a * HID:(a + 1) * HID, a * HD:(a + 1) * HD] = (
                    wout_ref[g * PACK + a].astype(jnp.bfloat16))
        lane = lax.broadcasted_iota(jnp.int32, (N, N * HD), 1) // HD
        sub = lax.broadcasted_iota(jnp.int32, (N, N * HD), 0)
        emat_s[:] = (lane == sub).astype(jnp.bfloat16)

    xb = x_ref[0].astype(jnp.bfloat16)

    zt = lax.dot_general(rw_ref[:].astype(jnp.bfloat16), xb, _TT,
                         preferred_element_type=jnp.float32)
    znt = zt / jnp.maximum(jnp.sqrt(jnp.sum(zt * zt, axis=0, keepdims=True)), 1e-12)
    c = c_ref[:]
    cn = c / jnp.maximum(jnp.sqrt(jnp.sum(c * c, axis=1, keepdims=True)), 1e-12)
    lt = lax.dot_general(cn.astype(jnp.bfloat16), znt.astype(jnp.bfloat16),
                         _NT, preferred_element_type=jnp.float32)
    lt = lt * (jnp.exp(temp_ref[0]) / math.sqrt(G))

    nidx = lax.broadcasted_iota(jnp.int32, (N, TM), 0)
    neg = jnp.float32(-1e30)
    m1 = jnp.max(lt, axis=0, keepdims=True)
    i1 = jnp.min(jnp.where(lt == m1, nidx, N), axis=0, keepdims=True)
    l2 = jnp.where(nidx == i1, neg, lt)
    m2 = jnp.max(l2, axis=0, keepdims=True)
    i2 = jnp.min(jnp.where(l2 == m2, nidx, N), axis=0, keepdims=True)

    e = jnp.exp(lt - m1)
    probs = e / jnp.sum(e, axis=0, keepdims=True)
    p1 = jnp.max(probs, axis=0, keepdims=True)
    p2 = jnp.max(jnp.where(nidx == i1, neg, probs), axis=0, keepdims=True)
    wt = jnp.where(nidx == i1, p1, 0.0) + jnp.where(nidx == i2, p2, 0.0)
    wexp = lax.dot_general(wt.astype(jnp.bfloat16), emat_s[:], _TN,
                           preferred_element_type=jnp.float32)

    hidden = jnp.dot(xb, win2_s[:], preferred_element_type=jnp.float32)
    hb = _gelu_tanh(hidden).astype(jnp.bfloat16)
    for g in range(N // PACK):
        out_ref[0, :, g * KP:(g + 1) * KP] = lax.dot_general(
            hb[:, g * KP:(g + 1) * KP], bd_s[g], _NT,
            preferred_element_type=jnp.float32) * wexp[:, g * KP:(g + 1) * KP]


def kernel(x, router_w, head_centroids, temperature, input_experts, output_experts):
    return pl.pallas_call(
        _body,
        grid=(T // TM,),
        in_specs=[
            pl.BlockSpec(memory_space=pltpu.SMEM),
            pl.BlockSpec((1, TM, D), lambda i: (i // SB, i % SB, 0)),
            pl.BlockSpec((G, D), lambda i: (0, 0)),
            pl.BlockSpec((N, G), lambda i: (0, 0)),
            pl.BlockSpec((N, D, HID), lambda i: (0, 0, 0)),
            pl.BlockSpec((N, HID, HD), lambda i: (0, 0, 0)),
        ],
        out_specs=pl.BlockSpec((1, TM, N * HD), lambda i: (i // SB, i % SB, 0)),
        out_shape=jax.ShapeDtypeStruct((B, S, N * HD), jnp.float32),
        scratch_shapes=[
            pltpu.VMEM((D, N * HID), jnp.bfloat16),
            pltpu.VMEM((N // PACK, KP, KP), jnp.bfloat16),
            pltpu.VMEM((N, N * HD), jnp.bfloat16),
        ],
        compiler_params=pltpu.CompilerParams(dimension_semantics=("arbitrary",), vmem_limit_bytes=100 * 1024 * 1024),
    )(temperature, x, router_w, head_centroids, input_experts, output_experts)


# speedup vs baseline: 1.2092x; 1.0612x over previous
import math

import jax
import jax.numpy as jnp
from jax import lax
from jax.experimental import pallas as pl
from jax.experimental.pallas import tpu as pltpu

B, S, D = 2, 2048, 1024
N, HID, HD, G = 16, 64, 64, 64
T = B * S
TM = 1024
SB = S // TM
PACK = 4
KP = PACK * HID

_TT = (((1,), (1,)), ((), ()))
_NT = (((1,), (0,)), ((), ()))
_TN = (((0,), (0,)), ((), ()))


def _gelu_tanh(h):
    u = h * (jnp.float32(0.7978845608028654)
             + jnp.float32(0.7978845608028654 * 0.044715) * (h * h))
    r = 0.5 * h
    return r + r * jnp.tanh(u)


def _body(temp_ref, x_ref, rw_ref, c_ref, win_ref, wout_ref, out_ref,
          win2_s, bd_s, emat_s):
    @pl.when(pl.program_id(0) == 0)
    def _prep():
        for n in range(N):
            win2_s[:, n * HID:(n + 1) * HID] = win_ref[n].astype(jnp.bfloat16)
        bd_s[:] = jnp.zeros((N // PACK, KP, KP), jnp.bfloat16)
        for g in range(N // PACK):
            for a in range(PACK):
                bd_s[g, a * HID:(a + 1) * HID, a * HD:(a + 1) * HD] = (
                    wout_ref[g * PACK + a].astype(jnp.bfloat16))
        lane = lax.broadcasted_iota(jnp.int32, (N, N * HD), 1) // HD
        sub = lax.broadcasted_iota(jnp.int32, (N, N * HD), 0)
        emat_s[:] = (lane == sub).astype(jnp.bfloat16)

    xb = x_ref[0].astype(jnp.bfloat16)

    zt = lax.dot_general(rw_ref[:].astype(jnp.bfloat16), xb, _TT,
                         preferred_element_type=jnp.float32)
    znt = zt / jnp.maximum(jnp.sqrt(jnp.sum(zt * zt, axis=0, keepdims=True)), 1e-12)
    c = c_ref[:]
    cn = c / jnp.maximum(jnp.sqrt(jnp.sum(c * c, axis=1, keepdims=True)), 1e-12)
    lt = lax.dot_general(cn.astype(jnp.bfloat16), znt.astype(jnp.bfloat16),
                         _NT, preferred_element_type=jnp.float32)
    lt = lt * (jnp.exp(temp_ref[0]) / math.sqrt(G))

    nidx = lax.broadcasted_iota(jnp.int32, (N, TM), 0)
    neg = jnp.float32(-1e30)
    m1 = jnp.max(lt, axis=0, keepdims=True)
    i1 = jnp.min(jnp.where(lt == m1, nidx, N), axis=0, keepdims=True)
    l2 = jnp.where(nidx == i1, neg, lt)
    m2 = jnp.max(l2, axis=0, keepdims=True)
    i2 = jnp.min(jnp.where(l2 == m2, nidx, N), axis=0, keepdims=True)

    e = jnp.exp(lt - m1)
    probs = e / jnp.sum(e, axis=0, keepdims=True)
    p1 = jnp.max(probs, axis=0, keepdims=True)
    p2 = jnp.max(jnp.where(nidx == i1, neg, probs), axis=0, keepdims=True)
    wt = jnp.where(nidx == i1, p1, 0.0) + jnp.where(nidx == i2, p2, 0.0)
    wexp = lax.dot_general(wt.astype(jnp.bfloat16), emat_s[:], _TN,
                           preferred_element_type=jnp.float32)

    hidden = jnp.dot(xb, win2_s[:], preferred_element_type=jnp.float32)
    hb = _gelu_tanh(hidden).astype(jnp.bfloat16)
    for g in range(N // PACK):
        out_ref[0, :, g * KP:(g + 1) * KP] = lax.dot_general(
            hb[:, g * KP:(g + 1) * KP], bd_s[g], _NT,
            preferred_element_type=jnp.float32) * wexp[:, g * KP:(g + 1) * KP]


def kernel(x, router_w, head_centroids, temperature, input_experts, output_experts):
    return pl.pallas_call(
        _body,
        grid=(T // TM,),
        in_specs=[
            pl.BlockSpec(memory_space=pltpu.SMEM),
            pl.BlockSpec((1, TM, D), lambda i: (i // SB, i % SB, 0)),
            pl.BlockSpec((G, D), lambda i: (0, 0)),
            pl.BlockSpec((N, G), lambda i: (0, 0)),
            pl.BlockSpec((N, D, HID), lambda i: (0, 0, 0)),
            pl.BlockSpec((N, HID, HD), lambda i: (0, 0, 0)),
        ],
        out_specs=pl.BlockSpec((1, TM, N * HD), lambda i: (i // SB, i % SB, 0)),
        out_shape=jax.ShapeDtypeStruct((B, S, N * HD), jnp.float32),
        scratch_shapes=[
            pltpu.VMEM((D, N * HID), jnp.bfloat16),
            pltpu.VMEM((N // PACK, KP, KP), jnp.bfloat16),
            pltpu.VMEM((N, N * HD), jnp.bfloat16),
        ],
        compiler_params=pltpu.CompilerParams(dimension_semantics=("arbitrary",)),
    )(temperature, x, router_w, head_centroids, input_experts, output_experts)
